# baseline (speedup 1.0000x reference)
import jax
import jax.numpy as jnp
from jax.experimental import pallas as pl
from jax.experimental.pallas import tpu as pltpu

TH = 0.85
BI = 512
NB = 8


def _qlayer_kern(x_ref, w_ref, b_ref, out_ref, rxs_ref, wsum_ref, tacc_ref):
    bi = pl.program_id(0)
    f = x_ref.shape[1]

    @pl.when(bi == 0)
    def _prep():
        xaq = x_ref[:].astype(jnp.bfloat16).astype(jnp.float32)
        rxs_ref[0, :] = jnp.sum(xaq, axis=1)
        wq = w_ref[:].astype(jnp.bfloat16).astype(jnp.float32)
        wsum_ref[0, :] = jnp.sum(wq, axis=1)
        tacc_ref[0, :] = jnp.zeros_like(tacc_ref[0, :])

    xb = x_ref[pl.ds(bi * BI, BI), :]
    rxs_b = rxs_ref[0, pl.ds(bi * BI, BI)]

    gram_d = jnp.dot(xb, xb.T, preferred_element_type=jnp.float32)
    c_d = gram_d * gram_d >= TH
    rr_d = jnp.sum(jnp.where(c_d, rxs_b[None, :], 0.0), axis=1)
    sq = jnp.sum(xb * xb, axis=1)
    diag_c = (sq * sq >= TH).astype(jnp.float32)
    t0 = rr_d - diag_c * rxs_b

    def pair_body(bj, t_loc):
        xj = x_ref[pl.ds(bj * BI, BI), :]
        gram = jnp.dot(xb, xj.T, preferred_element_type=jnp.float32)
        c = gram * gram >= TH
        rxs_j = rxs_ref[0, pl.ds(bj * BI, BI)]
        t_loc = t_loc + jnp.sum(jnp.where(c, rxs_j[None, :], 0.0), axis=1)
        cr = jnp.sum(jnp.where(c, rxs_b[:, None], 0.0), axis=0)
        tacc_ref[0, pl.ds(bj * BI, BI)] += cr
        return t_loc

    t_loc = jax.lax.fori_loop(bi + 1, NB, pair_body, t0)
    t = t_loc + tacc_ref[0, pl.ds(bi * BI, BI)]
    scalar = (t / f).astype(jnp.bfloat16).astype(jnp.float32)
    out_ref[:] = scalar[:, None] * wsum_ref[0, :][None, :] + b_ref[0, :][None, :]


@jax.jit
def kernel(x, W, b):
    n, f = x.shape
    h = W.shape[0]
    b2 = b.reshape(1, h)
    return pl.pallas_call(
        _qlayer_kern,
        grid=(n // BI,),
        in_specs=[
            pl.BlockSpec((n, f), lambda i: (0, 0)),
            pl.BlockSpec((h, f), lambda i: (0, 0)),
            pl.BlockSpec((1, h), lambda i: (0, 0)),
        ],
        out_specs=pl.BlockSpec((BI, h), lambda i: (i, 0)),
        out_shape=jax.ShapeDtypeStruct((n, h), jnp.float32),
        scratch_shapes=[
            pltpu.VMEM((1, n), jnp.float32),
            pltpu.VMEM((1, h), jnp.float32),
            pltpu.VMEM((1, n), jnp.float32),
        ],
    )(x, W, b2)


# speedup vs baseline: 1.2706x; 1.2706x over previous
import jax
import jax.numpy as jnp
from jax.experimental import pallas as pl
from jax.experimental.pallas import tpu as pltpu

TH = 0.85
BI = 512
NCORE = 2


def _qlayer_kern(x_blk_ref, x_ref, w_ref, b_ref, out_ref, rxs_ref, wsum_ref):
    k = pl.program_id(1)
    xb = x_blk_ref[:]
    n = x_ref.shape[0]
    f = x_ref.shape[1]

    @pl.when(k == 0)
    def _prep():
        xaq = x_ref[:].astype(jnp.bfloat16).astype(jnp.float32)
        rxs_ref[0, :] = jnp.sum(xaq, axis=1)
        wq = w_ref[:].astype(jnp.bfloat16).astype(jnp.float32)
        wsum_ref[0, :] = jnp.sum(wq, axis=1)

    c_id = pl.program_id(0)
    bi = c_id * (n // (BI * NCORE)) + k
    gram = jnp.dot(xb, x_ref[:].T, preferred_element_type=jnp.float32)
    c = gram * gram >= TH
    rxs = rxs_ref[0, :]
    t_d = jnp.sum(jnp.where(c, rxs[None, :], 0.0), axis=1)
    sq = jnp.sum(xb * xb, axis=1)
    diag_c = (sq * sq >= TH).astype(jnp.float32)
    rxs_b = rxs_ref[0, pl.ds(bi * BI, BI)]
    t = t_d - diag_c * rxs_b
    scalar = (t / f).astype(jnp.bfloat16).astype(jnp.float32)
    out_ref[:] = scalar[:, None] * wsum_ref[0, :][None, :] + b_ref[0, :][None, :]


@jax.jit
def kernel(x, W, b):
    n, f = x.shape
    h = W.shape[0]
    nk = n // (BI * NCORE)
    b2 = b.reshape(1, h)
    return pl.pallas_call(
        _qlayer_kern,
        grid=(NCORE, nk),
        in_specs=[
            pl.BlockSpec((BI, f), lambda c, k: (c * nk + k, 0)),
            pl.BlockSpec((n, f), lambda c, k: (0, 0)),
            pl.BlockSpec((h, f), lambda c, k: (0, 0)),
            pl.BlockSpec((1, h), lambda c, k: (0, 0)),
        ],
        out_specs=pl.BlockSpec((BI, h), lambda c, k: (c * nk + k, 0)),
        out_shape=jax.ShapeDtypeStruct((n, h), jnp.float32),
        scratch_shapes=[
            pltpu.VMEM((1, n), jnp.float32),
            pltpu.VMEM((1, h), jnp.float32),
        ],
        compiler_params=pltpu.CompilerParams(
            dimension_semantics=("parallel", "arbitrary"),
        ),
    )(x, x, W, b2)


# speedup vs baseline: 1.4167x; 1.1149x over previous
import jax
import jax.numpy as jnp
from jax.experimental import pallas as pl
from jax.experimental.pallas import tpu as pltpu

TH = 0.85
BI = 1024


def _qlayer_kern(x_blk_ref, x_ref, w_ref, b_ref, out_ref, rxs_ref, wsum_ref):
    i = pl.program_id(0)
    xb = x_blk_ref[:]
    f = x_ref.shape[1]

    @pl.when(i == 0)
    def _prep():
        xaq = x_ref[:].astype(jnp.bfloat16).astype(jnp.float32)
        rxs_ref[0, :] = jnp.sum(xaq, axis=1)
        wq = w_ref[:].astype(jnp.bfloat16).astype(jnp.float32)
        wsum_ref[0, :] = jnp.sum(wq, axis=1)

    gram = jnp.dot(xb, x_ref[:].T, preferred_element_type=jnp.float32)
    c = gram * gram >= TH
    rxs = rxs_ref[0, :]
    t_d = jnp.sum(jnp.where(c, rxs[None, :], 0.0), axis=1)
    sq = jnp.sum(xb * xb, axis=1)
    diag_c = (sq * sq >= TH).astype(jnp.float32)
    rxs_b = rxs_ref[0, pl.ds(i * BI, BI)]
    t = t_d - diag_c * rxs_b
    scalar = (t / f).astype(jnp.bfloat16).astype(jnp.float32)
    out_ref[:] = scalar[:, None] * wsum_ref[0, :][None, :] + b_ref[0, :][None, :]


@jax.jit
def kernel(x, W, b):
    n, f = x.shape
    h = W.shape[0]
    b2 = b.reshape(1, h)
    return pl.pallas_call(
        _qlayer_kern,
        grid=(n // BI,),
        in_specs=[
            pl.BlockSpec((BI, f), lambda i: (i, 0)),
            pl.BlockSpec((n, f), lambda i: (0, 0)),
            pl.BlockSpec((h, f), lambda i: (0, 0)),
            pl.BlockSpec((1, h), lambda i: (0, 0)),
        ],
        out_specs=pl.BlockSpec((BI, h), lambda i: (i, 0)),
        out_shape=jax.ShapeDtypeStruct((n, h), jnp.float32),
        scratch_shapes=[
            pltpu.VMEM((1, n), jnp.float32),
            pltpu.VMEM((1, h), jnp.float32),
        ],
    )(x, x, W, b2)


# speedup vs baseline: 1.4638x; 1.0333x over previous
import jax
import jax.numpy as jnp
from jax.experimental import pallas as pl
from jax.experimental.pallas import tpu as pltpu

TH = 0.85
BI = 1024


def _qlayer_kern(x_blk_ref, x_ref, w_ref, b_ref, out_ref,
                 rxs_row_ref, rxs_col_ref, wsum_ref):
    i = pl.program_id(0)
    xb = x_blk_ref[:]
    f = x_ref.shape[1]

    @pl.when(i == 0)
    def _prep():
        xa = x_ref[:]
        ones_row = jnp.ones((1, f), dtype=jnp.float32)
        ones_col = jnp.ones((f, 1), dtype=jnp.float32)
        rxs_row_ref[:] = jnp.dot(ones_row, xa.T,
                                 preferred_element_type=jnp.float32)
        rxs_col_ref[:] = jnp.dot(xa, ones_col,
                                 preferred_element_type=jnp.float32)
        wq = w_ref[:].astype(jnp.bfloat16).astype(jnp.float32)
        wsum_ref[:] = jnp.sum(wq, axis=1)[None, :]

    gram = jnp.dot(xb, x_ref[:].T, preferred_element_type=jnp.float32)
    c = gram * gram >= TH
    rxs = rxs_row_ref[:]
    t_d = jnp.sum(jnp.where(c, rxs, 0.0), axis=1, keepdims=True)
    ones_col = jnp.ones((f, 1), dtype=jnp.float32)
    sq = jnp.dot(xb * xb, ones_col, preferred_element_type=jnp.float32)
    diag_c = (sq * sq >= TH).astype(jnp.float32)
    rxs_b = rxs_col_ref[pl.ds(i * BI, BI), :]
    t = t_d - diag_c * rxs_b
    scalar = (t / f).astype(jnp.bfloat16).astype(jnp.float32)
    out_ref[:] = scalar * wsum_ref[:] + b_ref[:]


@jax.jit
def kernel(x, W, b):
    n, f = x.shape
    h = W.shape[0]
    b2 = b.reshape(1, h)
    return pl.pallas_call(
        _qlayer_kern,
        grid=(n // BI,),
        in_specs=[
            pl.BlockSpec((BI, f), lambda i: (i, 0)),
            pl.BlockSpec((n, f), lambda i: (0, 0)),
            pl.BlockSpec((h, f), lambda i: (0, 0)),
            pl.BlockSpec((1, h), lambda i: (0, 0)),
        ],
        out_specs=pl.BlockSpec((BI, h), lambda i: (i, 0)),
        out_shape=jax.ShapeDtypeStruct((n, h), jnp.float32),
        scratch_shapes=[
            pltpu.VMEM((1, n), jnp.float32),
            pltpu.VMEM((n, 1), jnp.float32),
            pltpu.VMEM((1, h), jnp.float32),
        ],
    )(x, x, W, b2)


# speedup vs baseline: 1.7843x; 1.2189x over previous
import jax
import jax.numpy as jnp
from jax.experimental import pallas as pl

TH = 0.85
BI = 1024
NB = 4


def _qlayer_kern(x_ref, w_ref, b_ref, out_ref):
    f = x_ref.shape[1]
    xa = x_ref[:]
    ones_row = jnp.ones((1, f), dtype=jnp.float32)
    ones_col = jnp.ones((f, 1), dtype=jnp.float32)
    rxs_row = jnp.dot(ones_row, xa.T, preferred_element_type=jnp.float32)
    rxs_col = jnp.dot(xa, ones_col, preferred_element_type=jnp.float32)
    wq = w_ref[:].astype(jnp.bfloat16).astype(jnp.float32)
    wsum = jnp.sum(wq, axis=1)[None, :]

    xblk = [xa[bi * BI:(bi + 1) * BI, :] for bi in range(NB)]
    rrow = [rxs_row[:, bi * BI:(bi + 1) * BI] for bi in range(NB)]
    rcol = [rxs_col[bi * BI:(bi + 1) * BI, :] for bi in range(NB)]

    t = [None] * NB
    tcol = [None] * NB
    for bi in range(NB):
        xb = xblk[bi]
        gram = jnp.dot(xb, xb.T, preferred_element_type=jnp.float32)
        c = gram * gram >= TH
        acc = jnp.sum(jnp.where(c, rrow[bi], 0.0), axis=1, keepdims=True)
        sq = jnp.dot(xb * xb, ones_col, preferred_element_type=jnp.float32)
        diag_c = (sq * sq >= TH).astype(jnp.float32)
        t[bi] = acc - diag_c * rcol[bi]
        for bj in range(bi + 1, NB):
            gram = jnp.dot(xb, xblk[bj].T, preferred_element_type=jnp.float32)
            c = gram * gram >= TH
            t[bi] = t[bi] + jnp.sum(jnp.where(c, rrow[bj], 0.0),
                                    axis=1, keepdims=True)
            cr = jnp.sum(jnp.where(c, rcol[bi], 0.0), axis=0, keepdims=True)
            tcol[bj] = cr if tcol[bj] is None else tcol[bj] + cr

    for bi in range(NB):
        tt = t[bi] if tcol[bi] is None else t[bi] + tcol[bi].reshape(BI, 1)
        scalar = (tt / f).astype(jnp.bfloat16).astype(jnp.float32)
        out_ref[bi * BI:(bi + 1) * BI, :] = scalar * wsum + b_ref[:]


@jax.jit
def kernel(x, W, b):
    n, f = x.shape
    h = W.shape[0]
    b2 = b.reshape(1, h)
    return pl.pallas_call(
        _qlayer_kern,
        out_shape=jax.ShapeDtypeStruct((n, h), jnp.float32),
    )(x, W, b2)
